# baseline (speedup 1.0000x reference)
import functools

import numpy as np
import jax
import jax.numpy as jnp
from jax import lax
from jax.experimental import pallas as pl
from jax.experimental.pallas import tpu as pltpu
from jax.experimental.pallas import tpu_sc as plsc

D_MODEL = 64
MAX_POS = 128
SCALE = 8.0

NUM_CORES = 2
NUM_SUBCORES = 16
NUM_WORKERS = NUM_CORES * NUM_SUBCORES
CHUNK = 128
NBUF = 4


def _pos_encoding_np():
    position = np.arange(MAX_POS)[:, np.newaxis]
    k = np.arange(D_MODEL)[np.newaxis, :]
    i = k // 2
    angle_rates = 1 / np.power(10000, 2 * i / np.float32(D_MODEL))
    angle_rads = position * angle_rates
    angle_rads[:, 0::2] = np.sin(angle_rads[:, 0::2])
    angle_rads[:, 1::2] = np.cos(angle_rads[:, 1::2])
    return angle_rads.astype(np.float32)


_POS = _pos_encoding_np()


@functools.partial(jax.jit, static_argnames=("n_seq",))
def _sc_embed(x2d, pos, table, *, n_seq):
    seq_per_w = n_seq // NUM_WORKERS
    n_rounds = seq_per_w // NBUF
    n_rows = n_seq * MAX_POS

    mesh = plsc.VectorSubcoreMesh(core_axis_name="c", subcore_axis_name="s")

    @functools.partial(
        pl.kernel,
        mesh=mesh,
        compiler_params=pltpu.CompilerParams(use_tc_tiling_on_sc=False),
        out_type=jax.ShapeDtypeStruct((n_rows, D_MODEL), jnp.float32),
        scratch_types=(
            [pltpu.VMEM((seq_per_w, CHUNK), jnp.int32)]
            + [pltpu.VMEM((MAX_POS, D_MODEL), jnp.float32)]
            + [pltpu.VMEM((CHUNK, D_MODEL), jnp.float32)] * NBUF
            + [pltpu.VMEM((CHUNK, D_MODEL), jnp.float32)] * NBUF
            + [pltpu.SemaphoreType.DMA] * (2 * NBUF)
        ),
    )
    def k(x_hbm, pos_hbm, table_hbm, out_hbm, idx_v, pos_v, *bufs):
        rows = bufs[:NBUF]
        outs = bufs[NBUF:2 * NBUF]
        gsem = bufs[2 * NBUF:3 * NBUF]
        osem = bufs[3 * NBUF:4 * NBUF]

        wid = lax.axis_index("s") * NUM_CORES + lax.axis_index("c")
        w_seq = wid * seq_per_w
        pltpu.sync_copy(pos_hbm, pos_v)
        pltpu.sync_copy(x_hbm.at[pl.ds(w_seq, seq_per_w), :], idx_v)

        def gather_start(g, b):
            pltpu.async_copy(table_hbm.at[idx_v.at[g]], rows[b], gsem[b])

        def out_slot(g):
            return out_hbm.at[pl.ds((w_seq + g) * CHUNK, CHUNK)]

        for b in range(NBUF):
            gather_start(b, b)

        def round_body(o, carry):
            for b in range(NBUF):
                g = o * NBUF + b
                pltpu.make_async_copy(
                    table_hbm.at[idx_v.at[g]], rows[b], gsem[b]).wait()

                @pl.when(o > 0)
                def _():
                    pltpu.make_async_copy(outs[b], out_slot(g), osem[b]).wait()

                def row_body(r, c2):
                    for c in range(D_MODEL // 16):
                        sl = pl.ds(c * 16, 16)
                        outs[b][r, sl] = rows[b][r, sl] * SCALE + pos_v[r, sl]
                    return c2

                lax.fori_loop(0, CHUNK, row_body, 0, unroll=2)

                @pl.when(o < n_rounds - 1)
                def _():
                    gather_start(g + NBUF, b)

                pltpu.async_copy(outs[b], out_slot(g), osem[b])
            return carry

        lax.fori_loop(0, n_rounds, round_body, 0, unroll=False)

        for b in range(NBUF):
            g = (n_rounds - 1) * NBUF + b
            pltpu.make_async_copy(outs[b], out_slot(g), osem[b]).wait()

    return k(x2d, pos, table)


def kernel(x, table):
    b, s = x.shape
    pos = jnp.asarray(_POS)
    out = _sc_embed(x, pos, table, n_seq=b)
    return out.reshape(b, s, D_MODEL)
